# baseline (speedup 1.0000x reference)
import functools

import jax
import jax.numpy as jnp
from jax import lax
from jax.experimental import pallas as pl
from jax.experimental.pallas import tpu as pltpu
from jax.experimental.pallas import tpu_sc as plsc

_K = 16
_POS_PAD = 128
_MHI = -65536

_BF = jnp.bfloat16


def _dot_bf16(a, b):
    return jnp.dot(a.astype(_BF), b.astype(_BF),
                   preferred_element_type=jnp.float32)


def _ln(xb, g, b, eps=1e-5):
    m = jnp.mean(xb, axis=-1, keepdims=True)
    v = jnp.mean((xb - m) ** 2, axis=-1, keepdims=True)
    return (xb - m) / jnp.sqrt(v + eps) * g + b


def _pack_bf16_pair(a):
    h = a.shape[1] // 2
    ai = lax.bitcast_convert_type(a, jnp.int32)
    hi = ai[:, 0:h]
    lo = ai[:, h:2 * h]

    def rne(w):
        return (w + jnp.int32(0x7FFF) + ((w >> 16) & 1)) & jnp.int32(_MHI)

    packed = rne(hi) | ((rne(lo) >> 16) & jnp.int32(0xFFFF))
    return lax.bitcast_convert_type(packed, jnp.float32)


def _qkv_body(x_ref, pos_ref, gq_ref, bq_ref, gkv_ref, bkv_ref,
              wq_ref, wk_ref, wv_ref, q_ref, tbl_ref):
    xb = x_ref[0]
    posb = pos_ref[0]
    xq = _ln(xb, gq_ref[...], bq_ref[...])
    xkv = _ln(xb, gkv_ref[...], bkv_ref[...])
    q = _dot_bf16(xq, wq_ref[...])
    k = _dot_bf16(xkv, wk_ref[...])
    v = _dot_bf16(xkv, wv_ref[...])
    pad = jnp.zeros((posb.shape[0], _POS_PAD - posb.shape[1]), jnp.float32)
    q_ref[0] = q
    tbl_ref[0] = jnp.concatenate(
        [_pack_bf16_pair(k), _pack_bf16_pair(v), posb, pad], axis=1)


def _knn_body(pos_ref, posT_ref, idx_ref, *, blk, n, k, row_base, idx_off):
    nb = pl.program_id(0)
    pb = pos_ref[...]
    pT = posT_ref[...]
    a2r = jnp.sum(pb * pb, axis=1, keepdims=True)
    a2c = jnp.sum(pT * pT, axis=0, keepdims=True)
    d = a2r + a2c - 2.0 * _dot_bf16(pb, pT)
    d = jnp.maximum(d, 0.0)
    cols = lax.broadcasted_iota(jnp.int32, (blk, n), 1)
    rows = row_base + nb * blk + lax.broadcasted_iota(jnp.int32, (blk, n), 0)
    d = jnp.where(cols == rows, 0.0, d)
    big_i = jnp.int32(1 << 30)
    inf = jnp.float32(jnp.inf)
    outs = []
    for _ in range(k):
        m = jnp.min(d, axis=1, keepdims=True)
        cand = jnp.where(d == m, cols, big_i)
        j = jnp.min(cand, axis=1, keepdims=True)
        outs.append(j)
        d = jnp.where(cand == j, inf, d)
    idx_ref[...] = jnp.concatenate(outs, axis=1) + idx_off


def _attn_body(q_ref, g_ref, pos_ref, w1_ref, b1_ref, w2t_ref, b2_ref,
               wp_ref, bp_ref, out_ref, *, c, k, scale):
    q = q_ref[...]
    posb = pos_ref[...]
    b1 = b1_ref[...]
    w2s = jnp.sum(w2t_ref[...].astype(_BF).astype(jnp.float32),
                  axis=0, keepdims=True)
    b2s = jnp.sum(b2_ref[...])
    w1r = [w1_ref[i:i + 1, :].astype(_BF).astype(jnp.float32)
           for i in range(3)]
    hc = c // 2
    q_hi = q[:, 0:hc]
    q_lo = q[:, hc:c]
    mhi = jnp.int32(_MHI)

    def unpack(words):
        wi = lax.bitcast_convert_type(words, jnp.int32)
        a_hi = lax.bitcast_convert_type(wi & mhi, jnp.float32)
        a_lo = lax.bitcast_convert_type(wi << 16, jnp.float32)
        return a_hi, a_lo

    logits = []
    for nidx in range(k):
        k_hi, k_lo = unpack(g_ref[nidx, :, 0:hc])
        qk = jnp.sum(q_hi * k_hi + q_lo * k_lo, axis=1, keepdims=True)
        pn = g_ref[nidx, :, c:c + 3]
        rel = (posb - pn).astype(_BF).astype(jnp.float32)
        h = b1
        for ci in range(3):
            h = h + rel[:, ci:ci + 1] * w1r[ci]
        h = jnp.maximum(h, 0.0).astype(_BF).astype(jnp.float32)
        rpe = jnp.sum(h * w2s, axis=1, keepdims=True) + b2s
        logits.append((qk + rpe) * scale)
    logits = jnp.concatenate(logits, axis=1)
    mx = jnp.max(logits, axis=1, keepdims=True)
    e = jnp.exp(logits - mx)
    s = jnp.sum(e, axis=1, keepdims=True)
    attn = e / s
    attn = jnp.where(jnp.isnan(attn), 0.0, attn)
    acc_hi = jnp.zeros((q.shape[0], hc), jnp.float32)
    acc_lo = jnp.zeros((q.shape[0], hc), jnp.float32)
    for nidx in range(k):
        v_hi, v_lo = unpack(g_ref[nidx, :, hc:c])
        a_n = attn[:, nidx:nidx + 1]
        acc_hi = acc_hi + a_n * v_hi
        acc_lo = acc_lo + a_n * v_lo
    acc = jnp.concatenate([acc_hi, acc_lo], axis=1)
    out_ref[...] = _dot_bf16(acc, wp_ref[...]) + bp_ref[...]


def _build_gather(tot, width):
    info = plsc.get_sparse_core_info()
    nc, ns = info.num_cores, info.num_subcores
    nw = nc * ns
    per_w = tot // nw
    chunk = 128
    n_chunks = per_w // chunk
    assert n_chunks % 2 == 0
    mesh = plsc.VectorSubcoreMesh(core_axis_name="c", subcore_axis_name="s")

    @functools.partial(
        pl.kernel, mesh=mesh,
        out_type=jax.ShapeDtypeStruct((tot, width), jnp.float32),
        scratch_types=[
            pltpu.VMEM((per_w,), jnp.int32),
            pltpu.VMEM((chunk, width), jnp.float32),
            pltpu.VMEM((chunk, width), jnp.float32),
            pltpu.SemaphoreType.DMA,
            pltpu.SemaphoreType.DMA,
            pltpu.SemaphoreType.DMA,
            pltpu.SemaphoreType.DMA,
        ],
    )
    def gather_kernel(tbl_hbm, idx_hbm, out_hbm, idx_v,
                      rows0, rows1, sg0, sg1, sw0, sw1):
        wid = lax.axis_index("s") * nc + lax.axis_index("c")
        base = wid * per_w
        rows = (rows0, rows1)
        sg = (sg0, sg1)
        sw = (sw0, sw1)

        pltpu.sync_copy(idx_hbm.at[pl.ds(base, per_w)], idx_v)

        def g_desc(i, b):
            return pltpu.make_async_copy(
                tbl_hbm.at[idx_v.at[pl.ds(i * chunk, chunk)]], rows[b], sg[b])

        def w_desc(i, b):
            return pltpu.make_async_copy(
                rows[b], out_hbm.at[pl.ds(base + i * chunk, chunk)], sw[b])

        g_desc(0, 0).start()

        def body(i2, carry):
            for bb in range(2):
                i = i2 * 2 + bb
                g_desc(i, bb).wait()
                w_desc(i, bb).start()

                @pl.when(i + 1 < n_chunks)
                def _():
                    @pl.when(i >= 1)
                    def _():
                        w_desc(i - 1, 1 - bb).wait()
                    g_desc(i + 1, 1 - bb).start()
            return carry

        lax.fori_loop(0, n_chunks // 2, body, 0)
        w_desc(n_chunks - 2, 0).wait()
        w_desc(n_chunks - 1, 1).wait()

    return gather_kernel


def _neighbor_gather(tbl2, idx_t, width):
    return _build_gather(idx_t.shape[0], width)(tbl2, idx_t)


def kernel(x, pos, gq, bq, gkv, bkv, Wq, Wk, Wv, W1, b1, W2, b2, Wp, bp):
    B, N, C = x.shape
    scale = C ** (-0.5)
    tblw = C + _POS_PAD

    blk_a = 512
    q, tbl = pl.pallas_call(
        _qkv_body,
        grid=(B, N // blk_a),
        in_specs=[
            pl.BlockSpec((1, blk_a, C), lambda b, i: (b, i, 0)),
            pl.BlockSpec((1, blk_a, 3), lambda b, i: (b, i, 0)),
            pl.BlockSpec((1, C), lambda b, i: (0, 0)),
            pl.BlockSpec((1, C), lambda b, i: (0, 0)),
            pl.BlockSpec((1, C), lambda b, i: (0, 0)),
            pl.BlockSpec((1, C), lambda b, i: (0, 0)),
            pl.BlockSpec((C, C), lambda b, i: (0, 0)),
            pl.BlockSpec((C, C), lambda b, i: (0, 0)),
            pl.BlockSpec((C, C), lambda b, i: (0, 0)),
        ],
        out_specs=[
            pl.BlockSpec((1, blk_a, C), lambda b, i: (b, i, 0)),
            pl.BlockSpec((1, blk_a, tblw), lambda b, i: (b, i, 0)),
        ],
        out_shape=[
            jax.ShapeDtypeStruct((B, N, C), jnp.float32),
            jax.ShapeDtypeStruct((B, N, tblw), jnp.float32),
        ],
    )(x, pos, gq.reshape(1, C), bq.reshape(1, C), gkv.reshape(1, C),
      bkv.reshape(1, C), Wq, Wk, Wv)

    halves = 2
    rows_c = N // halves
    blk_b = 128
    blk_c = 128
    rpe = W1.shape[1]
    posT = jnp.swapaxes(pos, 1, 2)
    tbl2 = tbl.reshape(B * N, tblw)
    q2 = q.reshape(B * N, C)
    pos2 = pos.reshape(B * N, 3)
    b1r = b1.reshape(1, -1)
    w2t = W2.T
    b2r = b2.reshape(1, C)
    bpr = bp.reshape(1, C)

    outs = []
    for b in range(B):
        idx_b = pl.pallas_call(
            functools.partial(_knn_body, blk=blk_b, n=N, k=_K,
                              row_base=0, idx_off=b * N),
            grid=(N // blk_b,),
            in_specs=[
                pl.BlockSpec((blk_b, 3), lambda i: (i, 0)),
                pl.BlockSpec((3, N), lambda i: (0, 0)),
            ],
            out_specs=pl.BlockSpec((blk_b, _K), lambda i: (i, 0)),
            out_shape=jax.ShapeDtypeStruct((N, _K), jnp.int32),
        )(pos[b], posT[b])
        for h in range(halves):
            row0 = h * rows_c
            idx_c = lax.slice_in_dim(idx_b, row0, row0 + rows_c)

            idx_t = idx_c.T.reshape(-1)
            g = _neighbor_gather(tbl2, idx_t, tblw)
            G = g.reshape(_K, rows_c, tblw)

            base = b * N + row0
            out_c = pl.pallas_call(
                functools.partial(_attn_body, c=C, k=_K, scale=scale),
                grid=(rows_c // blk_c,),
                in_specs=[
                    pl.BlockSpec((blk_c, C), lambda i: (i, 0)),
                    pl.BlockSpec((_K, blk_c, tblw), lambda i: (0, i, 0)),
                    pl.BlockSpec((blk_c, 3), lambda i: (i, 0)),
                    pl.BlockSpec((3, rpe), lambda i: (0, 0)),
                    pl.BlockSpec((1, rpe), lambda i: (0, 0)),
                    pl.BlockSpec((C, rpe), lambda i: (0, 0)),
                    pl.BlockSpec((1, C), lambda i: (0, 0)),
                    pl.BlockSpec((C, C), lambda i: (0, 0)),
                    pl.BlockSpec((1, C), lambda i: (0, 0)),
                ],
                out_specs=pl.BlockSpec((blk_c, C), lambda i: (i, 0)),
                out_shape=jax.ShapeDtypeStruct((rows_c, C), jnp.float32),
            )(lax.slice_in_dim(q2, base, base + rows_c), G,
              lax.slice_in_dim(pos2, base, base + rows_c), W1,
              b1r, w2t, b2r, Wp, bpr)
            outs.append(out_c)

    return jnp.concatenate(outs, axis=0).reshape(B, N, C)


# speedup vs baseline: 1.1078x; 1.1078x over previous
import functools

import jax
import jax.numpy as jnp
from jax import lax
from jax.experimental import pallas as pl
from jax.experimental.pallas import tpu as pltpu
from jax.experimental.pallas import tpu_sc as plsc

_K = 16
_POS_PAD = 128
_MHI = -65536

_BF = jnp.bfloat16


def _dot_bf16(a, b):
    return jnp.dot(a.astype(_BF), b.astype(_BF),
                   preferred_element_type=jnp.float32)


def _ln(xb, g, b, eps=1e-5):
    m = jnp.mean(xb, axis=-1, keepdims=True)
    v = jnp.mean((xb - m) ** 2, axis=-1, keepdims=True)
    return (xb - m) / jnp.sqrt(v + eps) * g + b


def _pack_bf16_pair(a):
    h = a.shape[1] // 2
    ai = lax.bitcast_convert_type(a, jnp.int32)
    hi = ai[:, 0:h]
    lo = ai[:, h:2 * h]

    def rne(w):
        return (w + jnp.int32(0x7FFF) + ((w >> 16) & 1)) & jnp.int32(_MHI)

    packed = rne(hi) | ((rne(lo) >> 16) & jnp.int32(0xFFFF))
    return lax.bitcast_convert_type(packed, jnp.float32)


def _qkv_body(x_ref, pos_ref, gq_ref, bq_ref, gkv_ref, bkv_ref,
              wq_ref, wk_ref, wv_ref, q_ref, tbl_ref):
    xb = x_ref[0]
    posb = pos_ref[0]
    xq = _ln(xb, gq_ref[...], bq_ref[...])
    xkv = _ln(xb, gkv_ref[...], bkv_ref[...])
    q = _dot_bf16(xq, wq_ref[...])
    k = _dot_bf16(xkv, wk_ref[...])
    v = _dot_bf16(xkv, wv_ref[...])
    pad = jnp.zeros((posb.shape[0], _POS_PAD - posb.shape[1]), jnp.float32)
    q_ref[0] = q
    tbl_ref[0] = jnp.concatenate(
        [_pack_bf16_pair(k), _pack_bf16_pair(v), posb, pad], axis=1)


def _knn_body(pos_ref, posT_ref, idx_ref, *, blk, n, k, row_base, idx_off):
    nb = pl.program_id(0)
    pb = pos_ref[...]
    pT = posT_ref[...]
    a2r = jnp.sum(pb * pb, axis=1, keepdims=True)
    a2c = jnp.sum(pT * pT, axis=0, keepdims=True)
    d = a2r + a2c - 2.0 * _dot_bf16(pb, pT)
    d = jnp.maximum(d, 0.0)
    cols = lax.broadcasted_iota(jnp.int32, (blk, n), 1)
    rows = row_base + nb * blk + lax.broadcasted_iota(jnp.int32, (blk, n), 0)
    d = jnp.where(cols == rows, 0.0, d)
    big_i = jnp.int32(1 << 30)
    inf = jnp.float32(jnp.inf)
    outs = []
    for _ in range(k):
        m = jnp.min(d, axis=1, keepdims=True)
        cand = jnp.where(d == m, cols, big_i)
        j = jnp.min(cand, axis=1, keepdims=True)
        outs.append(j)
        d = jnp.where(cand == j, inf, d)
    idx_ref[...] = jnp.concatenate(outs, axis=1) + idx_off


def _attn_body(q_ref, g_ref, pos_ref, w1_ref, b1_ref, w2t_ref, b2_ref,
               wp_ref, bp_ref, out_ref, *, c, k, scale):
    q = q_ref[...]
    posb = pos_ref[...]
    b1 = b1_ref[...]
    w2s = jnp.sum(w2t_ref[...].astype(_BF).astype(jnp.float32),
                  axis=0, keepdims=True)
    b2s = jnp.sum(b2_ref[...])
    w1r = [w1_ref[i:i + 1, :].astype(_BF).astype(jnp.float32)
           for i in range(3)]
    hc = c // 2
    q_hi = q[:, 0:hc]
    q_lo = q[:, hc:c]
    mhi = jnp.int32(_MHI)

    def unpack(words):
        wi = lax.bitcast_convert_type(words, jnp.int32)
        a_hi = lax.bitcast_convert_type(wi & mhi, jnp.float32)
        a_lo = lax.bitcast_convert_type(wi << 16, jnp.float32)
        return a_hi, a_lo

    logits = []
    for nidx in range(k):
        k_hi, k_lo = unpack(g_ref[nidx, :, 0:hc])
        qk = jnp.sum(q_hi * k_hi + q_lo * k_lo, axis=1, keepdims=True)
        pn = g_ref[nidx, :, c:c + 3]
        rel = (posb - pn).astype(_BF).astype(jnp.float32)
        h = b1
        for ci in range(3):
            h = h + rel[:, ci:ci + 1] * w1r[ci]
        h = jnp.maximum(h, 0.0).astype(_BF).astype(jnp.float32)
        rpe = jnp.sum(h * w2s, axis=1, keepdims=True) + b2s
        logits.append((qk + rpe) * scale)
    logits = jnp.concatenate(logits, axis=1)
    mx = jnp.max(logits, axis=1, keepdims=True)
    e = jnp.exp(logits - mx)
    s = jnp.sum(e, axis=1, keepdims=True)
    attn = e / s
    attn = jnp.where(jnp.isnan(attn), 0.0, attn)
    acc_hi = jnp.zeros((q.shape[0], hc), jnp.float32)
    acc_lo = jnp.zeros((q.shape[0], hc), jnp.float32)
    for nidx in range(k):
        v_hi, v_lo = unpack(g_ref[nidx, :, hc:c])
        a_n = attn[:, nidx:nidx + 1]
        acc_hi = acc_hi + a_n * v_hi
        acc_lo = acc_lo + a_n * v_lo
    acc = jnp.concatenate([acc_hi, acc_lo], axis=1)
    out_ref[...] = _dot_bf16(acc, wp_ref[...]) + bp_ref[...]


def _build_gather(tot, width):
    info = plsc.get_sparse_core_info()
    nc, ns = info.num_cores, info.num_subcores
    nw = nc * ns
    per_w = tot // nw
    chunk = 128
    n_chunks = per_w // chunk
    assert n_chunks % 2 == 0
    mesh = plsc.VectorSubcoreMesh(core_axis_name="c", subcore_axis_name="s")

    @functools.partial(
        pl.kernel, mesh=mesh,
        out_type=jax.ShapeDtypeStruct((tot, width), jnp.float32),
        scratch_types=[
            pltpu.VMEM((per_w,), jnp.int32),
            pltpu.VMEM((chunk, width), jnp.float32),
            pltpu.VMEM((chunk, width), jnp.float32),
            pltpu.SemaphoreType.DMA,
            pltpu.SemaphoreType.DMA,
            pltpu.SemaphoreType.DMA,
            pltpu.SemaphoreType.DMA,
        ],
    )
    def gather_kernel(tbl_hbm, idx_hbm, out_hbm, idx_v,
                      rows0, rows1, sg0, sg1, sw0, sw1):
        wid = lax.axis_index("s") * nc + lax.axis_index("c")
        base = wid * per_w
        rows = (rows0, rows1)
        sg = (sg0, sg1)
        sw = (sw0, sw1)

        pltpu.sync_copy(idx_hbm.at[pl.ds(base, per_w)], idx_v)

        def g_desc(i, b):
            return pltpu.make_async_copy(
                tbl_hbm.at[idx_v.at[pl.ds(i * chunk, chunk)]], rows[b], sg[b])

        def w_desc(i, b):
            return pltpu.make_async_copy(
                rows[b], out_hbm.at[pl.ds(base + i * chunk, chunk)], sw[b])

        g_desc(0, 0).start()

        def body(i2, carry):
            for bb in range(2):
                i = i2 * 2 + bb
                g_desc(i, bb).wait()
                w_desc(i, bb).start()

                @pl.when(i + 1 < n_chunks)
                def _():
                    @pl.when(i >= 1)
                    def _():
                        w_desc(i - 1, 1 - bb).wait()
                    g_desc(i + 1, 1 - bb).start()
            return carry

        lax.fori_loop(0, n_chunks // 2, body, 0)
        w_desc(n_chunks - 2, 0).wait()
        w_desc(n_chunks - 1, 1).wait()

    return gather_kernel


def _neighbor_gather(tbl2, idx_t, width):
    return _build_gather(idx_t.shape[0], width)(tbl2, idx_t)


def kernel(x, pos, gq, bq, gkv, bkv, Wq, Wk, Wv, W1, b1, W2, b2, Wp, bp):
    B, N, C = x.shape
    scale = C ** (-0.5)
    tblw = C + _POS_PAD

    blk_a = 512
    q, tbl = pl.pallas_call(
        _qkv_body,
        grid=(B, N // blk_a),
        in_specs=[
            pl.BlockSpec((1, blk_a, C), lambda b, i: (b, i, 0)),
            pl.BlockSpec((1, blk_a, 3), lambda b, i: (b, i, 0)),
            pl.BlockSpec((1, C), lambda b, i: (0, 0)),
            pl.BlockSpec((1, C), lambda b, i: (0, 0)),
            pl.BlockSpec((1, C), lambda b, i: (0, 0)),
            pl.BlockSpec((1, C), lambda b, i: (0, 0)),
            pl.BlockSpec((C, C), lambda b, i: (0, 0)),
            pl.BlockSpec((C, C), lambda b, i: (0, 0)),
            pl.BlockSpec((C, C), lambda b, i: (0, 0)),
        ],
        out_specs=[
            pl.BlockSpec((1, blk_a, C), lambda b, i: (b, i, 0)),
            pl.BlockSpec((1, blk_a, tblw), lambda b, i: (b, i, 0)),
        ],
        out_shape=[
            jax.ShapeDtypeStruct((B, N, C), jnp.float32),
            jax.ShapeDtypeStruct((B, N, tblw), jnp.float32),
        ],
    )(x, pos, gq.reshape(1, C), bq.reshape(1, C), gkv.reshape(1, C),
      bkv.reshape(1, C), Wq, Wk, Wv)

    halves = 2
    rows_c = N // halves
    blk_b = 256
    blk_c = 256
    rpe = W1.shape[1]
    posT = jnp.swapaxes(pos, 1, 2)
    tbl2 = tbl.reshape(B * N, tblw)
    q2 = q.reshape(B * N, C)
    pos2 = pos.reshape(B * N, 3)
    b1r = b1.reshape(1, -1)
    w2t = W2.T
    b2r = b2.reshape(1, C)
    bpr = bp.reshape(1, C)

    outs = []
    for b in range(B):
        idx_b = pl.pallas_call(
            functools.partial(_knn_body, blk=blk_b, n=N, k=_K,
                              row_base=0, idx_off=b * N),
            grid=(N // blk_b,),
            in_specs=[
                pl.BlockSpec((blk_b, 3), lambda i: (i, 0)),
                pl.BlockSpec((3, N), lambda i: (0, 0)),
            ],
            out_specs=pl.BlockSpec((blk_b, _K), lambda i: (i, 0)),
            out_shape=jax.ShapeDtypeStruct((N, _K), jnp.int32),
        )(pos[b], posT[b])
        for h in range(halves):
            row0 = h * rows_c
            idx_c = lax.slice_in_dim(idx_b, row0, row0 + rows_c)

            idx_t = idx_c.T.reshape(-1)
            g = _neighbor_gather(tbl2, idx_t, tblw)
            G = g.reshape(_K, rows_c, tblw)

            base = b * N + row0
            out_c = pl.pallas_call(
                functools.partial(_attn_body, c=C, k=_K, scale=scale),
                grid=(rows_c // blk_c,),
                in_specs=[
                    pl.BlockSpec((blk_c, C), lambda i: (i, 0)),
                    pl.BlockSpec((_K, blk_c, tblw), lambda i: (0, i, 0)),
                    pl.BlockSpec((blk_c, 3), lambda i: (i, 0)),
                    pl.BlockSpec((3, rpe), lambda i: (0, 0)),
                    pl.BlockSpec((1, rpe), lambda i: (0, 0)),
                    pl.BlockSpec((C, rpe), lambda i: (0, 0)),
                    pl.BlockSpec((1, C), lambda i: (0, 0)),
                    pl.BlockSpec((C, C), lambda i: (0, 0)),
                    pl.BlockSpec((1, C), lambda i: (0, 0)),
                ],
                out_specs=pl.BlockSpec((blk_c, C), lambda i: (i, 0)),
                out_shape=jax.ShapeDtypeStruct((rows_c, C), jnp.float32),
            )(lax.slice_in_dim(q2, base, base + rows_c), G,
              lax.slice_in_dim(pos2, base, base + rows_c), W1,
              b1r, w2t, b2r, Wp, bpr)
            outs.append(out_c)

    return jnp.concatenate(outs, axis=0).reshape(B, N, C)


# speedup vs baseline: 1.1149x; 1.0064x over previous
import functools

import jax
import jax.numpy as jnp
from jax import lax
from jax.experimental import pallas as pl
from jax.experimental.pallas import tpu as pltpu
from jax.experimental.pallas import tpu_sc as plsc

_K = 16
_POS_PAD = 128
_MHI = -65536

_BF = jnp.bfloat16


def _dot_bf16(a, b):
    return jnp.dot(a.astype(_BF), b.astype(_BF),
                   preferred_element_type=jnp.float32)


def _ln(xb, g, b, eps=1e-5):
    m = jnp.mean(xb, axis=-1, keepdims=True)
    v = jnp.mean((xb - m) ** 2, axis=-1, keepdims=True)
    return (xb - m) / jnp.sqrt(v + eps) * g + b


def _pack_bf16_pair(a):
    h = a.shape[1] // 2
    ai = lax.bitcast_convert_type(a, jnp.int32)
    hi = ai[:, 0:h]
    lo = ai[:, h:2 * h]

    def rne(w):
        return (w + jnp.int32(0x7FFF) + ((w >> 16) & 1)) & jnp.int32(_MHI)

    packed = rne(hi) | ((rne(lo) >> 16) & jnp.int32(0xFFFF))
    return lax.bitcast_convert_type(packed, jnp.float32)


def _qkv_body(x_ref, pos_ref, gq_ref, bq_ref, gkv_ref, bkv_ref,
              wq_ref, wk_ref, wv_ref, q_ref, tbl_ref):
    xb = x_ref[0]
    posb = pos_ref[0]
    xq = _ln(xb, gq_ref[...], bq_ref[...])
    xkv = _ln(xb, gkv_ref[...], bkv_ref[...])
    q = _dot_bf16(xq, wq_ref[...])
    k = _dot_bf16(xkv, wk_ref[...])
    v = _dot_bf16(xkv, wv_ref[...])
    pad = jnp.zeros((posb.shape[0], _POS_PAD - posb.shape[1]), jnp.float32)
    q_ref[0] = q
    tbl_ref[0] = jnp.concatenate(
        [_pack_bf16_pair(k), _pack_bf16_pair(v), posb, pad], axis=1)


def _knn_body(pos_ref, posT_ref, idx_ref, *, blk, n, k, row_base, idx_off):
    nb = pl.program_id(0)
    pb = pos_ref[...]
    pT = posT_ref[...]
    a2r = jnp.sum(pb * pb, axis=1, keepdims=True)
    a2c = jnp.sum(pT * pT, axis=0, keepdims=True)
    d = a2r + a2c - 2.0 * _dot_bf16(pb, pT)
    d = jnp.maximum(d, 0.0)
    cols = lax.broadcasted_iota(jnp.int32, (blk, n), 1)
    rows = row_base + nb * blk + lax.broadcasted_iota(jnp.int32, (blk, n), 0)
    d = jnp.where(cols == rows, 0.0, d)
    big_i = jnp.int32(1 << 30)
    inf = jnp.float32(jnp.inf)
    outs = []
    for _ in range(k):
        m = jnp.min(d, axis=1, keepdims=True)
        cand = jnp.where(d == m, cols, big_i)
        j = jnp.min(cand, axis=1, keepdims=True)
        outs.append(j)
        d = jnp.where(cand == j, inf, d)
    idx_ref[...] = jnp.concatenate(outs, axis=1) + idx_off


def _attn_body(q_ref, g_ref, pos_ref, w1_ref, b1_ref, w2t_ref, b2_ref,
               wp_ref, bp_ref, out_ref, *, c, k, scale):
    q = q_ref[...]
    posb = pos_ref[...]
    b1 = b1_ref[...]
    w2s = jnp.sum(w2t_ref[...].astype(_BF).astype(jnp.float32),
                  axis=0, keepdims=True)
    b2s = jnp.sum(b2_ref[...])
    w1r = [w1_ref[i:i + 1, :].astype(_BF).astype(jnp.float32)
           for i in range(3)]
    hc = c // 2
    q_hi = q[:, 0:hc]
    q_lo = q[:, hc:c]
    mhi = jnp.int32(_MHI)

    def unpack(words):
        wi = lax.bitcast_convert_type(words, jnp.int32)
        a_hi = lax.bitcast_convert_type(wi & mhi, jnp.float32)
        a_lo = lax.bitcast_convert_type(wi << 16, jnp.float32)
        return a_hi, a_lo

    logits = []
    for nidx in range(k):
        k_hi, k_lo = unpack(g_ref[nidx, :, 0:hc])
        qk = jnp.sum(q_hi * k_hi + q_lo * k_lo, axis=1, keepdims=True)
        pn = g_ref[nidx, :, c:c + 3]
        rel = (posb - pn).astype(_BF).astype(jnp.float32)
        h = b1
        for ci in range(3):
            h = h + rel[:, ci:ci + 1] * w1r[ci]
        h = jnp.maximum(h, 0.0).astype(_BF).astype(jnp.float32)
        rpe = jnp.sum(h * w2s, axis=1, keepdims=True) + b2s
        logits.append((qk + rpe) * scale)
    logits = jnp.concatenate(logits, axis=1)
    mx = jnp.max(logits, axis=1, keepdims=True)
    e = jnp.exp(logits - mx)
    s = jnp.sum(e, axis=1, keepdims=True)
    attn = e / s
    attn = jnp.where(jnp.isnan(attn), 0.0, attn)
    acc_hi = jnp.zeros((q.shape[0], hc), jnp.float32)
    acc_lo = jnp.zeros((q.shape[0], hc), jnp.float32)
    for nidx in range(k):
        v_hi, v_lo = unpack(g_ref[nidx, :, hc:c])
        a_n = attn[:, nidx:nidx + 1]
        acc_hi = acc_hi + a_n * v_hi
        acc_lo = acc_lo + a_n * v_lo
    acc = jnp.concatenate([acc_hi, acc_lo], axis=1)
    out_ref[...] = _dot_bf16(acc, wp_ref[...]) + bp_ref[...]


def _build_gather(tot, width):
    info = plsc.get_sparse_core_info()
    nc, ns = info.num_cores, info.num_subcores
    nw = nc * ns
    per_w = tot // nw
    chunk = 128
    n_chunks = per_w // chunk
    assert n_chunks % 2 == 0
    mesh = plsc.VectorSubcoreMesh(core_axis_name="c", subcore_axis_name="s")

    @functools.partial(
        pl.kernel, mesh=mesh,
        out_type=jax.ShapeDtypeStruct((tot, width), jnp.float32),
        scratch_types=[
            pltpu.VMEM((per_w,), jnp.int32),
            pltpu.VMEM((chunk, width), jnp.float32),
            pltpu.VMEM((chunk, width), jnp.float32),
            pltpu.SemaphoreType.DMA,
            pltpu.SemaphoreType.DMA,
            pltpu.SemaphoreType.DMA,
            pltpu.SemaphoreType.DMA,
        ],
    )
    def gather_kernel(tbl_hbm, idx_hbm, out_hbm, idx_v,
                      rows0, rows1, sg0, sg1, sw0, sw1):
        wid = lax.axis_index("s") * nc + lax.axis_index("c")
        base = wid * per_w
        rows = (rows0, rows1)
        sg = (sg0, sg1)
        sw = (sw0, sw1)

        pltpu.sync_copy(idx_hbm.at[pl.ds(base, per_w)], idx_v)

        def g_desc(i, b):
            return pltpu.make_async_copy(
                tbl_hbm.at[idx_v.at[pl.ds(i * chunk, chunk)]], rows[b], sg[b])

        def w_desc(i, b):
            return pltpu.make_async_copy(
                rows[b], out_hbm.at[pl.ds(base + i * chunk, chunk)], sw[b])

        g_desc(0, 0).start()

        def body(i2, carry):
            for bb in range(2):
                i = i2 * 2 + bb
                g_desc(i, bb).wait()
                w_desc(i, bb).start()

                @pl.when(i + 1 < n_chunks)
                def _():
                    @pl.when(i >= 1)
                    def _():
                        w_desc(i - 1, 1 - bb).wait()
                    g_desc(i + 1, 1 - bb).start()
            return carry

        lax.fori_loop(0, n_chunks // 2, body, 0)
        w_desc(n_chunks - 2, 0).wait()
        w_desc(n_chunks - 1, 1).wait()

    return gather_kernel


def _neighbor_gather(tbl2, idx_t, width):
    return _build_gather(idx_t.shape[0], width)(tbl2, idx_t)


def kernel(x, pos, gq, bq, gkv, bkv, Wq, Wk, Wv, W1, b1, W2, b2, Wp, bp):
    B, N, C = x.shape
    scale = C ** (-0.5)
    tblw = C + _POS_PAD

    blk_a = 512
    q, tbl = pl.pallas_call(
        _qkv_body,
        grid=(B, N // blk_a),
        in_specs=[
            pl.BlockSpec((1, blk_a, C), lambda b, i: (b, i, 0)),
            pl.BlockSpec((1, blk_a, 3), lambda b, i: (b, i, 0)),
            pl.BlockSpec((1, C), lambda b, i: (0, 0)),
            pl.BlockSpec((1, C), lambda b, i: (0, 0)),
            pl.BlockSpec((1, C), lambda b, i: (0, 0)),
            pl.BlockSpec((1, C), lambda b, i: (0, 0)),
            pl.BlockSpec((C, C), lambda b, i: (0, 0)),
            pl.BlockSpec((C, C), lambda b, i: (0, 0)),
            pl.BlockSpec((C, C), lambda b, i: (0, 0)),
        ],
        out_specs=[
            pl.BlockSpec((1, blk_a, C), lambda b, i: (b, i, 0)),
            pl.BlockSpec((1, blk_a, tblw), lambda b, i: (b, i, 0)),
        ],
        out_shape=[
            jax.ShapeDtypeStruct((B, N, C), jnp.float32),
            jax.ShapeDtypeStruct((B, N, tblw), jnp.float32),
        ],
    )(x, pos, gq.reshape(1, C), bq.reshape(1, C), gkv.reshape(1, C),
      bkv.reshape(1, C), Wq, Wk, Wv)

    halves = 2
    rows_c = N // halves
    blk_b = 512
    blk_c = 512
    rpe = W1.shape[1]
    posT = jnp.swapaxes(pos, 1, 2)
    tbl2 = tbl.reshape(B * N, tblw)
    q2 = q.reshape(B * N, C)
    pos2 = pos.reshape(B * N, 3)
    b1r = b1.reshape(1, -1)
    w2t = W2.T
    b2r = b2.reshape(1, C)
    bpr = bp.reshape(1, C)

    outs = []
    for b in range(B):
        idx_b = pl.pallas_call(
            functools.partial(_knn_body, blk=blk_b, n=N, k=_K,
                              row_base=0, idx_off=b * N),
            grid=(N // blk_b,),
            in_specs=[
                pl.BlockSpec((blk_b, 3), lambda i: (i, 0)),
                pl.BlockSpec((3, N), lambda i: (0, 0)),
            ],
            out_specs=pl.BlockSpec((blk_b, _K), lambda i: (i, 0)),
            out_shape=jax.ShapeDtypeStruct((N, _K), jnp.int32),
        )(pos[b], posT[b])
        for h in range(halves):
            row0 = h * rows_c
            idx_c = lax.slice_in_dim(idx_b, row0, row0 + rows_c)

            idx_t = idx_c.T.reshape(-1)
            g = _neighbor_gather(tbl2, idx_t, tblw)
            G = g.reshape(_K, rows_c, tblw)

            base = b * N + row0
            out_c = pl.pallas_call(
                functools.partial(_attn_body, c=C, k=_K, scale=scale),
                grid=(rows_c // blk_c,),
                in_specs=[
                    pl.BlockSpec((blk_c, C), lambda i: (i, 0)),
                    pl.BlockSpec((_K, blk_c, tblw), lambda i: (0, i, 0)),
                    pl.BlockSpec((blk_c, 3), lambda i: (i, 0)),
                    pl.BlockSpec((3, rpe), lambda i: (0, 0)),
                    pl.BlockSpec((1, rpe), lambda i: (0, 0)),
                    pl.BlockSpec((C, rpe), lambda i: (0, 0)),
                    pl.BlockSpec((1, C), lambda i: (0, 0)),
                    pl.BlockSpec((C, C), lambda i: (0, 0)),
                    pl.BlockSpec((1, C), lambda i: (0, 0)),
                ],
                out_specs=pl.BlockSpec((blk_c, C), lambda i: (i, 0)),
                out_shape=jax.ShapeDtypeStruct((rows_c, C), jnp.float32),
            )(lax.slice_in_dim(q2, base, base + rows_c), G,
              lax.slice_in_dim(pos2, base, base + rows_c), W1,
              b1r, w2t, b2r, Wp, bpr)
            outs.append(out_c)

    return jnp.concatenate(outs, axis=0).reshape(B, N, C)


# speedup vs baseline: 1.2110x; 1.0862x over previous
import functools

import jax
import jax.numpy as jnp
from jax import lax
from jax.experimental import pallas as pl
from jax.experimental.pallas import tpu as pltpu
from jax.experimental.pallas import tpu_sc as plsc

_K = 16
_POS_PAD = 128
_MHI = -65536

_BF = jnp.bfloat16


def _dot_bf16(a, b):
    return jnp.dot(a.astype(_BF), b.astype(_BF),
                   preferred_element_type=jnp.float32)


def _ln(xb, g, b, eps=1e-5):
    m = jnp.mean(xb, axis=-1, keepdims=True)
    v = jnp.mean((xb - m) ** 2, axis=-1, keepdims=True)
    return (xb - m) / jnp.sqrt(v + eps) * g + b


def _pack_bf16_pair(a):
    h = a.shape[1] // 2
    ai = lax.bitcast_convert_type(a, jnp.int32)
    hi = ai[:, 0:h]
    lo = ai[:, h:2 * h]

    def rne(w):
        return (w + jnp.int32(0x7FFF) + ((w >> 16) & 1)) & jnp.int32(_MHI)

    packed = rne(hi) | ((rne(lo) >> 16) & jnp.int32(0xFFFF))
    return lax.bitcast_convert_type(packed, jnp.float32)


def _qkv_body(x_ref, pos_ref, gq_ref, bq_ref, gkv_ref, bkv_ref,
              wq_ref, wk_ref, wv_ref, q_ref, tbl_ref):
    xb = x_ref[0]
    posb = pos_ref[0]
    xq = _ln(xb, gq_ref[...], bq_ref[...])
    xkv = _ln(xb, gkv_ref[...], bkv_ref[...])
    q = _dot_bf16(xq, wq_ref[...])
    k = _dot_bf16(xkv, wk_ref[...])
    v = _dot_bf16(xkv, wv_ref[...])
    pad = jnp.zeros((posb.shape[0], _POS_PAD - posb.shape[1]), jnp.float32)
    q_ref[0] = q
    tbl_ref[0] = jnp.concatenate(
        [_pack_bf16_pair(k), _pack_bf16_pair(v), posb, pad], axis=1)


def _knn_body(pos_ref, posT_ref, idx_ref, *, blk, n, k, row_base, idx_off):
    nb = pl.program_id(0)
    pb = pos_ref[...]
    pT = posT_ref[...]
    a2r = jnp.sum(pb * pb, axis=1, keepdims=True)
    a2c = jnp.sum(pT * pT, axis=0, keepdims=True)
    d = a2r + a2c - 2.0 * _dot_bf16(pb, pT)
    d = jnp.maximum(d, 0.0)
    cols = lax.broadcasted_iota(jnp.int32, (blk, n), 1)
    rows = row_base + nb * blk + lax.broadcasted_iota(jnp.int32, (blk, n), 0)
    d = jnp.where(cols == rows, 0.0, d)
    big_i = jnp.int32(1 << 30)
    inf = jnp.float32(jnp.inf)
    h = n // 2
    d_lo = d[:, 0:h]
    d_hi = d[:, h:n]
    colh = cols[:, 0:h]
    take_hi = d_hi < d_lo
    p = jnp.where(take_hi, d_hi, d_lo)
    qv = jnp.where(take_hi, d_lo, d_hi)
    jidx = jnp.where(take_hi, colh + h, colh)
    r2 = 2 * colh + h
    outs = []
    for _ in range(k):
        m = jnp.min(p, axis=1, keepdims=True)
        cand = jnp.where(p == m, jidx, big_i)
        j = jnp.min(cand, axis=1, keepdims=True)
        outs.append(j)
        hit = cand == j
        p = jnp.where(hit, qv, p)
        jidx = jnp.where(hit, r2 - jidx, jidx)
        qv = jnp.where(hit, inf, qv)
    idx_ref[...] = jnp.concatenate(outs, axis=1) + idx_off


def _attn_body(q_ref, g_ref, pos_ref, w1_ref, b1_ref, w2t_ref, b2_ref,
               wp_ref, bp_ref, out_ref, *, c, k, scale):
    q = q_ref[...]
    posb = pos_ref[...]
    b1 = b1_ref[...]
    w2s = jnp.sum(w2t_ref[...].astype(_BF).astype(jnp.float32),
                  axis=0, keepdims=True)
    b2s = jnp.sum(b2_ref[...])
    w1r = [w1_ref[i:i + 1, :].astype(_BF).astype(jnp.float32)
           for i in range(3)]
    hc = c // 2
    q_hi = q[:, 0:hc]
    q_lo = q[:, hc:c]
    mhi = jnp.int32(_MHI)

    def unpack(words):
        wi = lax.bitcast_convert_type(words, jnp.int32)
        a_hi = lax.bitcast_convert_type(wi & mhi, jnp.float32)
        a_lo = lax.bitcast_convert_type(wi << 16, jnp.float32)
        return a_hi, a_lo

    logits = []
    for nidx in range(k):
        k_hi, k_lo = unpack(g_ref[nidx, :, 0:hc])
        qk = jnp.sum(q_hi * k_hi + q_lo * k_lo, axis=1, keepdims=True)
        pn = g_ref[nidx, :, c:c + 3]
        rel = (posb - pn).astype(_BF).astype(jnp.float32)
        h = b1
        for ci in range(3):
            h = h + rel[:, ci:ci + 1] * w1r[ci]
        h = jnp.maximum(h, 0.0).astype(_BF).astype(jnp.float32)
        rpe = jnp.sum(h * w2s, axis=1, keepdims=True) + b2s
        logits.append((qk + rpe) * scale)
    logits = jnp.concatenate(logits, axis=1)
    mx = jnp.max(logits, axis=1, keepdims=True)
    e = jnp.exp(logits - mx)
    s = jnp.sum(e, axis=1, keepdims=True)
    attn = e / s
    attn = jnp.where(jnp.isnan(attn), 0.0, attn)
    acc_hi = jnp.zeros((q.shape[0], hc), jnp.float32)
    acc_lo = jnp.zeros((q.shape[0], hc), jnp.float32)
    for nidx in range(k):
        v_hi, v_lo = unpack(g_ref[nidx, :, hc:c])
        a_n = attn[:, nidx:nidx + 1]
        acc_hi = acc_hi + a_n * v_hi
        acc_lo = acc_lo + a_n * v_lo
    acc = jnp.concatenate([acc_hi, acc_lo], axis=1)
    out_ref[...] = _dot_bf16(acc, wp_ref[...]) + bp_ref[...]


def _build_gather(tot, width):
    info = plsc.get_sparse_core_info()
    nc, ns = info.num_cores, info.num_subcores
    nw = nc * ns
    per_w = tot // nw
    chunk = 128
    n_chunks = per_w // chunk
    assert n_chunks % 2 == 0
    mesh = plsc.VectorSubcoreMesh(core_axis_name="c", subcore_axis_name="s")

    @functools.partial(
        pl.kernel, mesh=mesh,
        out_type=jax.ShapeDtypeStruct((tot, width), jnp.float32),
        scratch_types=[
            pltpu.VMEM((per_w,), jnp.int32),
            pltpu.VMEM((chunk, width), jnp.float32),
            pltpu.VMEM((chunk, width), jnp.float32),
            pltpu.SemaphoreType.DMA,
            pltpu.SemaphoreType.DMA,
            pltpu.SemaphoreType.DMA,
            pltpu.SemaphoreType.DMA,
        ],
    )
    def gather_kernel(tbl_hbm, idx_hbm, out_hbm, idx_v,
                      rows0, rows1, sg0, sg1, sw0, sw1):
        wid = lax.axis_index("s") * nc + lax.axis_index("c")
        base = wid * per_w
        rows = (rows0, rows1)
        sg = (sg0, sg1)
        sw = (sw0, sw1)

        pltpu.sync_copy(idx_hbm.at[pl.ds(base, per_w)], idx_v)

        def g_desc(i, b):
            return pltpu.make_async_copy(
                tbl_hbm.at[idx_v.at[pl.ds(i * chunk, chunk)]], rows[b], sg[b])

        def w_desc(i, b):
            return pltpu.make_async_copy(
                rows[b], out_hbm.at[pl.ds(base + i * chunk, chunk)], sw[b])

        g_desc(0, 0).start()

        def body(i2, carry):
            for bb in range(2):
                i = i2 * 2 + bb
                g_desc(i, bb).wait()
                w_desc(i, bb).start()

                @pl.when(i + 1 < n_chunks)
                def _():
                    @pl.when(i >= 1)
                    def _():
                        w_desc(i - 1, 1 - bb).wait()
                    g_desc(i + 1, 1 - bb).start()
            return carry

        lax.fori_loop(0, n_chunks // 2, body, 0)
        w_desc(n_chunks - 2, 0).wait()
        w_desc(n_chunks - 1, 1).wait()

    return gather_kernel


def _neighbor_gather(tbl2, idx_t, width):
    return _build_gather(idx_t.shape[0], width)(tbl2, idx_t)


def kernel(x, pos, gq, bq, gkv, bkv, Wq, Wk, Wv, W1, b1, W2, b2, Wp, bp):
    B, N, C = x.shape
    scale = C ** (-0.5)
    tblw = C + _POS_PAD

    blk_a = 512
    q, tbl = pl.pallas_call(
        _qkv_body,
        grid=(B, N // blk_a),
        in_specs=[
            pl.BlockSpec((1, blk_a, C), lambda b, i: (b, i, 0)),
            pl.BlockSpec((1, blk_a, 3), lambda b, i: (b, i, 0)),
            pl.BlockSpec((1, C), lambda b, i: (0, 0)),
            pl.BlockSpec((1, C), lambda b, i: (0, 0)),
            pl.BlockSpec((1, C), lambda b, i: (0, 0)),
            pl.BlockSpec((1, C), lambda b, i: (0, 0)),
            pl.BlockSpec((C, C), lambda b, i: (0, 0)),
            pl.BlockSpec((C, C), lambda b, i: (0, 0)),
            pl.BlockSpec((C, C), lambda b, i: (0, 0)),
        ],
        out_specs=[
            pl.BlockSpec((1, blk_a, C), lambda b, i: (b, i, 0)),
            pl.BlockSpec((1, blk_a, tblw), lambda b, i: (b, i, 0)),
        ],
        out_shape=[
            jax.ShapeDtypeStruct((B, N, C), jnp.float32),
            jax.ShapeDtypeStruct((B, N, tblw), jnp.float32),
        ],
    )(x, pos, gq.reshape(1, C), bq.reshape(1, C), gkv.reshape(1, C),
      bkv.reshape(1, C), Wq, Wk, Wv)

    halves = 2
    rows_c = N // halves
    blk_b = 512
    blk_c = 512
    rpe = W1.shape[1]
    posT = jnp.swapaxes(pos, 1, 2)
    tbl2 = tbl.reshape(B * N, tblw)
    q2 = q.reshape(B * N, C)
    pos2 = pos.reshape(B * N, 3)
    b1r = b1.reshape(1, -1)
    w2t = W2.T
    b2r = b2.reshape(1, C)
    bpr = bp.reshape(1, C)

    outs = []
    for b in range(B):
        idx_b = pl.pallas_call(
            functools.partial(_knn_body, blk=blk_b, n=N, k=_K,
                              row_base=0, idx_off=b * N),
            grid=(N // blk_b,),
            in_specs=[
                pl.BlockSpec((blk_b, 3), lambda i: (i, 0)),
                pl.BlockSpec((3, N), lambda i: (0, 0)),
            ],
            out_specs=pl.BlockSpec((blk_b, _K), lambda i: (i, 0)),
            out_shape=jax.ShapeDtypeStruct((N, _K), jnp.int32),
        )(pos[b], posT[b])
        for h in range(halves):
            row0 = h * rows_c
            idx_c = lax.slice_in_dim(idx_b, row0, row0 + rows_c)

            idx_t = idx_c.T.reshape(-1)
            g = _neighbor_gather(tbl2, idx_t, tblw)
            G = g.reshape(_K, rows_c, tblw)

            base = b * N + row0
            out_c = pl.pallas_call(
                functools.partial(_attn_body, c=C, k=_K, scale=scale),
                grid=(rows_c // blk_c,),
                in_specs=[
                    pl.BlockSpec((blk_c, C), lambda i: (i, 0)),
                    pl.BlockSpec((_K, blk_c, tblw), lambda i: (0, i, 0)),
                    pl.BlockSpec((blk_c, 3), lambda i: (i, 0)),
                    pl.BlockSpec((3, rpe), lambda i: (0, 0)),
                    pl.BlockSpec((1, rpe), lambda i: (0, 0)),
                    pl.BlockSpec((C, rpe), lambda i: (0, 0)),
                    pl.BlockSpec((1, C), lambda i: (0, 0)),
                    pl.BlockSpec((C, C), lambda i: (0, 0)),
                    pl.BlockSpec((1, C), lambda i: (0, 0)),
                ],
                out_specs=pl.BlockSpec((blk_c, C), lambda i: (i, 0)),
                out_shape=jax.ShapeDtypeStruct((rows_c, C), jnp.float32),
            )(lax.slice_in_dim(q2, base, base + rows_c), G,
              lax.slice_in_dim(pos2, base, base + rows_c), W1,
              b1r, w2t, b2r, Wp, bpr)
            outs.append(out_c)

    return jnp.concatenate(outs, axis=0).reshape(B, N, C)


# speedup vs baseline: 1.2316x; 1.0170x over previous
import functools

import jax
import jax.numpy as jnp
from jax import lax
from jax.experimental import pallas as pl
from jax.experimental.pallas import tpu as pltpu
from jax.experimental.pallas import tpu_sc as plsc

_K = 16
_POS_PAD = 128
_MHI = -65536

_BF = jnp.bfloat16


def _dot_bf16(a, b):
    return jnp.dot(a.astype(_BF), b.astype(_BF),
                   preferred_element_type=jnp.float32)


def _ln(xb, g, b, eps=1e-5):
    m = jnp.mean(xb, axis=-1, keepdims=True)
    v = jnp.mean((xb - m) ** 2, axis=-1, keepdims=True)
    return (xb - m) / jnp.sqrt(v + eps) * g + b


def _pack_bf16_pair(a):
    h = a.shape[1] // 2
    ai = lax.bitcast_convert_type(a, jnp.int32)
    hi = ai[:, 0:h]
    lo = ai[:, h:2 * h]

    def rne(w):
        return (w + jnp.int32(0x7FFF) + ((w >> 16) & 1)) & jnp.int32(_MHI)

    packed = rne(hi) | ((rne(lo) >> 16) & jnp.int32(0xFFFF))
    return lax.bitcast_convert_type(packed, jnp.float32)


def _qkv_body(x_ref, pos_ref, gq_ref, bq_ref, gkv_ref, bkv_ref,
              wq_ref, wk_ref, wv_ref, q_ref, tbl_ref):
    xb = x_ref[0]
    posb = pos_ref[0]
    xq = _ln(xb, gq_ref[...], bq_ref[...])
    xkv = _ln(xb, gkv_ref[...], bkv_ref[...])
    q = _dot_bf16(xq, wq_ref[...])
    k = _dot_bf16(xkv, wk_ref[...])
    v = _dot_bf16(xkv, wv_ref[...])
    pad = jnp.zeros((posb.shape[0], _POS_PAD - posb.shape[1]), jnp.float32)
    q_ref[0] = q
    tbl_ref[0] = jnp.concatenate(
        [_pack_bf16_pair(k), _pack_bf16_pair(v), posb, pad], axis=1)


def _knn_body(pos_ref, posT_ref, idx_ref, *, blk, n, k, row_base, idx_off):
    nb = pl.program_id(0)
    pb = pos_ref[...]
    pT = posT_ref[...]
    a2r = jnp.sum(pb * pb, axis=1, keepdims=True)
    a2c = jnp.sum(pT * pT, axis=0, keepdims=True)
    d = a2r + a2c - 2.0 * _dot_bf16(pb, pT)
    d = jnp.maximum(d, 0.0)
    cols = lax.broadcasted_iota(jnp.int32, (blk, n), 1)
    rows = row_base + nb * blk + lax.broadcasted_iota(jnp.int32, (blk, n), 0)
    d = jnp.where(cols == rows, 0.0, d)
    big_i = jnp.int32(1 << 30)
    inf = jnp.float32(jnp.inf)
    h = n // 2
    d_lo = d[:, 0:h]
    d_hi = d[:, h:n]
    colh = cols[:, 0:h]
    take_hi = d_hi < d_lo
    p = jnp.where(take_hi, d_hi, d_lo)
    qv = jnp.where(take_hi, d_lo, d_hi)
    jidx = jnp.where(take_hi, colh + h, colh)
    r2 = 2 * colh + h
    outs = []
    for _ in range(k):
        m = jnp.min(p, axis=1, keepdims=True)
        cand = jnp.where(p == m, jidx, big_i)
        j = jnp.min(cand, axis=1, keepdims=True)
        outs.append(j)
        hit = cand == j
        p = jnp.where(hit, qv, p)
        jidx = jnp.where(hit, r2 - jidx, jidx)
        qv = jnp.where(hit, inf, qv)
    idx_ref[...] = jnp.concatenate(outs, axis=1) + idx_off


def _attn_body(q_ref, g_ref, pos_ref, w1_ref, b1_ref, w2t_ref, b2_ref,
               wp_ref, bp_ref, out_ref, *, c, k, scale):
    q = q_ref[...]
    posb = pos_ref[...]
    b1 = b1_ref[...]
    w2s = jnp.sum(w2t_ref[...].astype(_BF).astype(jnp.float32),
                  axis=0, keepdims=True)
    b2s = jnp.sum(b2_ref[...])
    w1r = [w1_ref[i:i + 1, :].astype(_BF).astype(jnp.float32)
           for i in range(3)]
    hc = c // 2
    q_hi = q[:, 0:hc]
    q_lo = q[:, hc:c]
    mhi = jnp.int32(_MHI)

    def unpack(words):
        wi = lax.bitcast_convert_type(words, jnp.int32)
        a_hi = lax.bitcast_convert_type(wi & mhi, jnp.float32)
        a_lo = lax.bitcast_convert_type(wi << 16, jnp.float32)
        return a_hi, a_lo

    logits = []
    for nidx in range(k):
        k_hi, k_lo = unpack(g_ref[nidx, :, 0:hc])
        qk = jnp.sum(q_hi * k_hi + q_lo * k_lo, axis=1, keepdims=True)
        pn = g_ref[nidx, :, c:c + 3]
        rel = (posb - pn).astype(_BF).astype(jnp.float32)
        h = b1
        for ci in range(3):
            h = h + rel[:, ci:ci + 1] * w1r[ci]
        h = jnp.maximum(h, 0.0).astype(_BF).astype(jnp.float32)
        rpe = jnp.sum(h * w2s, axis=1, keepdims=True) + b2s
        logits.append((qk + rpe) * scale)
    logits = jnp.concatenate(logits, axis=1)
    mx = jnp.max(logits, axis=1, keepdims=True)
    e = jnp.exp(logits - mx)
    s = jnp.sum(e, axis=1, keepdims=True)
    attn = e / s
    attn = jnp.where(jnp.isnan(attn), 0.0, attn)
    acc_hi = jnp.zeros((q.shape[0], hc), jnp.float32)
    acc_lo = jnp.zeros((q.shape[0], hc), jnp.float32)
    for nidx in range(k):
        v_hi, v_lo = unpack(g_ref[nidx, :, hc:c])
        a_n = attn[:, nidx:nidx + 1]
        acc_hi = acc_hi + a_n * v_hi
        acc_lo = acc_lo + a_n * v_lo
    acc = jnp.concatenate([acc_hi, acc_lo], axis=1)
    out_ref[...] = _dot_bf16(acc, wp_ref[...]) + bp_ref[...]


def _build_gather(tot, width):
    info = plsc.get_sparse_core_info()
    nc, ns = info.num_cores, info.num_subcores
    nw = nc * ns
    per_w = tot // nw
    chunk = 128
    n_chunks = per_w // chunk
    assert n_chunks % 2 == 0
    mesh = plsc.VectorSubcoreMesh(core_axis_name="c", subcore_axis_name="s")

    @functools.partial(
        pl.kernel, mesh=mesh,
        out_type=jax.ShapeDtypeStruct((tot, width), jnp.float32),
        scratch_types=[
            pltpu.VMEM((per_w,), jnp.int32),
            pltpu.VMEM((chunk, width), jnp.float32),
            pltpu.VMEM((chunk, width), jnp.float32),
            pltpu.SemaphoreType.DMA,
            pltpu.SemaphoreType.DMA,
            pltpu.SemaphoreType.DMA,
            pltpu.SemaphoreType.DMA,
        ],
    )
    def gather_kernel(tbl_hbm, idx_hbm, out_hbm, idx_v,
                      rows0, rows1, sg0, sg1, sw0, sw1):
        wid = lax.axis_index("s") * nc + lax.axis_index("c")
        base = wid * per_w
        rows = (rows0, rows1)
        sg = (sg0, sg1)
        sw = (sw0, sw1)

        pltpu.sync_copy(idx_hbm.at[pl.ds(base, per_w)], idx_v)

        def g_desc(i, b):
            return pltpu.make_async_copy(
                tbl_hbm.at[idx_v.at[pl.ds(i * chunk, chunk)]], rows[b], sg[b])

        def w_desc(i, b):
            return pltpu.make_async_copy(
                rows[b], out_hbm.at[pl.ds(base + i * chunk, chunk)], sw[b])

        g_desc(0, 0).start()

        def body(i2, carry):
            for bb in range(2):
                i = i2 * 2 + bb
                g_desc(i, bb).wait()
                w_desc(i, bb).start()

                @pl.when(i + 1 < n_chunks)
                def _():
                    @pl.when(i >= 1)
                    def _():
                        w_desc(i - 1, 1 - bb).wait()
                    g_desc(i + 1, 1 - bb).start()
            return carry

        lax.fori_loop(0, n_chunks // 2, body, 0)
        w_desc(n_chunks - 2, 0).wait()
        w_desc(n_chunks - 1, 1).wait()

    return gather_kernel


def _neighbor_gather(tbl2, idx_t, width):
    return _build_gather(idx_t.shape[0], width)(tbl2, idx_t)


def kernel(x, pos, gq, bq, gkv, bkv, Wq, Wk, Wv, W1, b1, W2, b2, Wp, bp):
    B, N, C = x.shape
    scale = C ** (-0.5)
    tblw = C + _POS_PAD

    blk_a = 512
    q, tbl = pl.pallas_call(
        _qkv_body,
        grid=(B, N // blk_a),
        in_specs=[
            pl.BlockSpec((1, blk_a, C), lambda b, i: (b, i, 0)),
            pl.BlockSpec((1, blk_a, 3), lambda b, i: (b, i, 0)),
            pl.BlockSpec((1, C), lambda b, i: (0, 0)),
            pl.BlockSpec((1, C), lambda b, i: (0, 0)),
            pl.BlockSpec((1, C), lambda b, i: (0, 0)),
            pl.BlockSpec((1, C), lambda b, i: (0, 0)),
            pl.BlockSpec((C, C), lambda b, i: (0, 0)),
            pl.BlockSpec((C, C), lambda b, i: (0, 0)),
            pl.BlockSpec((C, C), lambda b, i: (0, 0)),
        ],
        out_specs=[
            pl.BlockSpec((1, blk_a, C), lambda b, i: (b, i, 0)),
            pl.BlockSpec((1, blk_a, tblw), lambda b, i: (b, i, 0)),
        ],
        out_shape=[
            jax.ShapeDtypeStruct((B, N, C), jnp.float32),
            jax.ShapeDtypeStruct((B, N, tblw), jnp.float32),
        ],
    )(x, pos, gq.reshape(1, C), bq.reshape(1, C), gkv.reshape(1, C),
      bkv.reshape(1, C), Wq, Wk, Wv)

    halves = 1
    rows_c = N // halves
    blk_b = 512
    blk_c = 512
    rpe = W1.shape[1]
    posT = jnp.swapaxes(pos, 1, 2)
    tbl2 = tbl.reshape(B * N, tblw)
    q2 = q.reshape(B * N, C)
    pos2 = pos.reshape(B * N, 3)
    b1r = b1.reshape(1, -1)
    w2t = W2.T
    b2r = b2.reshape(1, C)
    bpr = bp.reshape(1, C)

    outs = []
    for b in range(B):
        idx_b = pl.pallas_call(
            functools.partial(_knn_body, blk=blk_b, n=N, k=_K,
                              row_base=0, idx_off=b * N),
            grid=(N // blk_b,),
            in_specs=[
                pl.BlockSpec((blk_b, 3), lambda i: (i, 0)),
                pl.BlockSpec((3, N), lambda i: (0, 0)),
            ],
            out_specs=pl.BlockSpec((blk_b, _K), lambda i: (i, 0)),
            out_shape=jax.ShapeDtypeStruct((N, _K), jnp.int32),
        )(pos[b], posT[b])
        for h in range(halves):
            row0 = h * rows_c
            idx_c = lax.slice_in_dim(idx_b, row0, row0 + rows_c)

            idx_t = idx_c.T.reshape(-1)
            g = _neighbor_gather(tbl2, idx_t, tblw)
            G = g.reshape(_K, rows_c, tblw)

            base = b * N + row0
            out_c = pl.pallas_call(
                functools.partial(_attn_body, c=C, k=_K, scale=scale),
                grid=(rows_c // blk_c,),
                in_specs=[
                    pl.BlockSpec((blk_c, C), lambda i: (i, 0)),
                    pl.BlockSpec((_K, blk_c, tblw), lambda i: (0, i, 0)),
                    pl.BlockSpec((blk_c, 3), lambda i: (i, 0)),
                    pl.BlockSpec((3, rpe), lambda i: (0, 0)),
                    pl.BlockSpec((1, rpe), lambda i: (0, 0)),
                    pl.BlockSpec((C, rpe), lambda i: (0, 0)),
                    pl.BlockSpec((1, C), lambda i: (0, 0)),
                    pl.BlockSpec((C, C), lambda i: (0, 0)),
                    pl.BlockSpec((1, C), lambda i: (0, 0)),
                ],
                out_specs=pl.BlockSpec((blk_c, C), lambda i: (i, 0)),
                out_shape=jax.ShapeDtypeStruct((rows_c, C), jnp.float32),
            )(lax.slice_in_dim(q2, base, base + rows_c), G,
              lax.slice_in_dim(pos2, base, base + rows_c), W1,
              b1r, w2t, b2r, Wp, bpr)
            outs.append(out_c)

    return jnp.concatenate(outs, axis=0).reshape(B, N, C)
